# Initial kernel scaffold; baseline (speedup 1.0000x reference)
import jax
import jax.numpy as jnp
from jax.experimental import pallas as pl


def kernel(inputs, split_values, split_index_array, estimator_weights, leaf_classes_array, features_by_estimator, path_identifier_list, internal_node_index_list):
    raise NotImplementedError("write your pallas kernel here")


# speedup vs baseline: 5.3999x; 5.3999x over previous
import functools

import jax
import jax.numpy as jnp
from jax import lax
from jax.experimental import pallas as pl
from jax.experimental.pallas import tpu as pltpu
from jax.experimental.pallas import tpu_sc as plsc

DEPTH = 5
E = 256
NVAR = 128
NSEL = 32
B = 1024
INTERNAL = 2 ** DEPTH - 1
LEAF = 2 ** DEPTH
LANES = 16
NWORKERS = 32
ROWS_PER_W = B // NWORKERS
NGROUPS = ROWS_PER_W // LANES


def _forest_sc(x, thr, fsel, ew, lc):
    mesh = plsc.VectorSubcoreMesh(core_axis_name="c", subcore_axis_name="s")

    @functools.partial(
        pl.kernel,
        out_type=jax.ShapeDtypeStruct((B,), jnp.float32),
        mesh=mesh,
        compiler_params=pltpu.CompilerParams(needs_layout_passes=False),
        scratch_types=[
            pltpu.VMEM((ROWS_PER_W, NVAR), jnp.float32),
            pltpu.VMEM((E * INTERNAL,), jnp.float32),
            pltpu.VMEM((E * INTERNAL,), jnp.int32),
            pltpu.VMEM((E * LEAF,), jnp.float32),
            pltpu.VMEM((E * LEAF,), jnp.float32),
            pltpu.VMEM((ROWS_PER_W,), jnp.float32),
        ],
    )
    def kern(x_hbm, thr_hbm, fsel_hbm, ew_hbm, lc_hbm, out_hbm,
             x_v, thr_v, fsel_v, ew_v, lc_v, res_v):
        wid = lax.axis_index("s") * 2 + lax.axis_index("c")
        base = wid * ROWS_PER_W
        pltpu.sync_copy(x_hbm.at[pl.ds(base, ROWS_PER_W)], x_v)
        pltpu.sync_copy(thr_hbm, thr_v)
        pltpu.sync_copy(fsel_hbm, fsel_v)
        pltpu.sync_copy(ew_hbm, ew_v)
        pltpu.sync_copy(lc_hbm, lc_v)

        iota = lax.broadcasted_iota(jnp.int32, (LANES,), 0)
        zero_i = jnp.zeros((LANES,), jnp.int32)
        one_i = jnp.ones((LANES,), jnp.int32)

        def body(e, carry):
            nxt = []
            nbase = e * INTERNAL
            lbase = e * LEAF
            for g in range(NGROUPS):
                m, s, a = carry[g]
                rows = iota + (g * LANES)
                node = zero_i
                leaf = zero_i
                for _ in range(DEPTH):
                    fi = plsc.load_gather(fsel_v, [nbase + node])
                    tv = plsc.load_gather(thr_v, [nbase + node])
                    xv = plsc.load_gather(x_v, [rows, fi])
                    u = lax.bitcast_convert_type(xv, jnp.int32)
                    u = u + 0x7FFF + (lax.shift_right_logical(u, 16) & 1)
                    xv = lax.bitcast_convert_type(u & jnp.int32(-65536),
                                                  jnp.float32)
                    diff = tv - xv
                    t = (diff / (1.0 + jnp.abs(diff)) + 1.0) / 2.0
                    bit = jnp.where(t > 0.5, zero_i, one_i)
                    leaf = leaf * 2 + bit
                    node = node * 2 + 1 + bit
                li = lbase + leaf
                w = plsc.load_gather(ew_v, [li])
                v = plsc.load_gather(lc_v, [li])
                mn = jnp.maximum(m, w)
                cs = jnp.exp(m - mn)
                p = jnp.exp(w - mn)
                nxt.append((mn, s * cs + p, a * cs + p * v))
            return tuple(nxt)

        init = tuple(
            (jnp.full((LANES,), -1e30, jnp.float32),
             jnp.zeros((LANES,), jnp.float32),
             jnp.zeros((LANES,), jnp.float32))
            for _ in range(NGROUPS)
        )
        fin = lax.fori_loop(0, E, body, init)
        for g in range(NGROUPS):
            mn, s, a = fin[g]
            res_v[pl.ds(g * LANES, LANES)] = a / s
        pltpu.sync_copy(res_v, out_hbm.at[pl.ds(base, ROWS_PER_W)])

    return kern(x, thr, fsel, ew, lc)


def kernel(inputs, split_values, split_index_array, estimator_weights,
           leaf_classes_array, features_by_estimator, path_identifier_list,
           internal_node_index_list):
    sia = jax.nn.softmax(split_index_array, axis=-1)
    n_star = jnp.argmax(sia, axis=-1).astype(jnp.int32)
    thr = jnp.take_along_axis(split_values, n_star[..., None], axis=-1)[..., 0]
    fsel = jnp.take_along_axis(features_by_estimator, n_star, axis=1)

    return _forest_sc(
        inputs,
        thr.reshape(-1),
        fsel.reshape(-1).astype(jnp.int32),
        estimator_weights.reshape(-1),
        leaf_classes_array.reshape(-1),
    )


# speedup vs baseline: 5.8875x; 1.0903x over previous
import functools

import jax
import jax.numpy as jnp
from jax import lax
from jax.experimental import pallas as pl
from jax.experimental.pallas import tpu as pltpu
from jax.experimental.pallas import tpu_sc as plsc

DEPTH = 5
E = 256
NVAR = 128
NSEL = 32
B = 1024
INTERNAL = 2 ** DEPTH - 1
LEAF = 2 ** DEPTH
LANES = 16
NWORKERS = 32
ROWS_PER_W = B // NWORKERS
NGROUPS = ROWS_PER_W // LANES
TWO_M24 = 5.9604644775390625e-08


def _forest_sc(x, thr, fsel, ew, lc):
    mesh = plsc.VectorSubcoreMesh(core_axis_name="c", subcore_axis_name="s")

    @functools.partial(
        pl.kernel,
        out_type=jax.ShapeDtypeStruct((B,), jnp.float32),
        mesh=mesh,
        compiler_params=pltpu.CompilerParams(needs_layout_passes=False),
        scratch_types=[
            pltpu.VMEM((ROWS_PER_W, NVAR), jnp.float32),
            pltpu.VMEM((E * INTERNAL,), jnp.float32),
            pltpu.VMEM((E * INTERNAL,), jnp.int32),
            pltpu.VMEM((E * LEAF,), jnp.float32),
            pltpu.VMEM((E * LEAF,), jnp.float32),
            pltpu.VMEM((ROWS_PER_W,), jnp.float32),
        ] + [pltpu.SemaphoreType.DMA] * 5,
    )
    def kern(x_hbm, thr_hbm, fsel_hbm, ew_hbm, lc_hbm, out_hbm,
             x_v, thr_v, fsel_v, ew_v, lc_v, res_v,
             sem0, sem1, sem2, sem3, sem4):
        wid = lax.axis_index("s") * 2 + lax.axis_index("c")
        base = wid * ROWS_PER_W
        cps = [
            pltpu.async_copy(x_hbm.at[pl.ds(base, ROWS_PER_W)], x_v, sem0),
            pltpu.async_copy(thr_hbm, thr_v, sem1),
            pltpu.async_copy(fsel_hbm, fsel_v, sem2),
            pltpu.async_copy(ew_hbm, ew_v, sem3),
            pltpu.async_copy(lc_hbm, lc_v, sem4),
        ]
        for cp in cps:
            cp.wait()

        iota = lax.broadcasted_iota(jnp.int32, (LANES,), 0)
        zero_i = jnp.zeros((LANES,), jnp.int32)
        one_i = jnp.ones((LANES,), jnp.int32)

        def body(e, carry):
            nxt = []
            lbase = e * LEAF
            for g in range(NGROUPS):
                s, a = carry[g]
                rows = iota + (g * LANES)
                leaf = zero_i
                for d in range(DEPTH):
                    idx = (e * INTERNAL + (2 ** d - 1)) + leaf
                    fi = plsc.load_gather(fsel_v, [idx])
                    tv = plsc.load_gather(thr_v, [idx])
                    xv = plsc.load_gather(x_v, [rows, fi])
                    u = lax.bitcast_convert_type(xv, jnp.int32)
                    u = u + 0x7FFF + (lax.shift_right_logical(u, 16) & 1)
                    xv = lax.bitcast_convert_type(u & jnp.int32(-65536),
                                                  jnp.float32)
                    diff = tv - xv
                    q = diff / (1.0 + jnp.abs(diff))
                    bit = jnp.where(q > TWO_M24, zero_i, one_i)
                    leaf = leaf + leaf + bit
                li = lbase + leaf
                w = plsc.load_gather(ew_v, [li])
                v = plsc.load_gather(lc_v, [li])
                p = jnp.exp(w)
                nxt.append((s + p, a + p * v))
            return tuple(nxt)

        init = tuple(
            (jnp.zeros((LANES,), jnp.float32),
             jnp.zeros((LANES,), jnp.float32))
            for _ in range(NGROUPS)
        )
        fin = lax.fori_loop(0, E, body, init)
        for g in range(NGROUPS):
            s, a = fin[g]
            res_v[pl.ds(g * LANES, LANES)] = a / s
        pltpu.sync_copy(res_v, out_hbm.at[pl.ds(base, ROWS_PER_W)])

    return kern(x, thr, fsel, ew, lc)


def kernel(inputs, split_values, split_index_array, estimator_weights,
           leaf_classes_array, features_by_estimator, path_identifier_list,
           internal_node_index_list):
    sia = jax.nn.softmax(split_index_array, axis=-1)
    n_star = jnp.argmax(sia, axis=-1).astype(jnp.int32)
    thr = jnp.take_along_axis(split_values, n_star[..., None], axis=-1)[..., 0]
    fsel = jnp.take_along_axis(features_by_estimator, n_star, axis=1)

    return _forest_sc(
        inputs,
        thr.reshape(-1),
        fsel.reshape(-1).astype(jnp.int32),
        estimator_weights.reshape(-1),
        leaf_classes_array.reshape(-1),
    )


# speedup vs baseline: 5.9320x; 1.0076x over previous
import functools

import jax
import jax.numpy as jnp
from jax import lax
from jax.experimental import pallas as pl
from jax.experimental.pallas import tpu as pltpu
from jax.experimental.pallas import tpu_sc as plsc

DEPTH = 5
E = 256
NVAR = 128
NSEL = 32
B = 1024
INTERNAL = 2 ** DEPTH - 1
LEAF = 2 ** DEPTH
LANES = 16
NWORKERS = 32
ROWS_PER_W = B // NWORKERS
NGROUPS = ROWS_PER_W // LANES
TWO_M24 = 5.9604644775390625e-08


def _forest_sc(x, thr, fsel, ew, lc):
    mesh = plsc.VectorSubcoreMesh(core_axis_name="c", subcore_axis_name="s")

    @functools.partial(
        pl.kernel,
        out_type=jax.ShapeDtypeStruct((B,), jnp.float32),
        mesh=mesh,
        compiler_params=pltpu.CompilerParams(needs_layout_passes=False),
        scratch_types=[
            pltpu.VMEM((ROWS_PER_W * NVAR,), jnp.float32),
            pltpu.VMEM((E * INTERNAL,), jnp.float32),
            pltpu.VMEM((E * INTERNAL,), jnp.int32),
            pltpu.VMEM((E * LEAF,), jnp.float32),
            pltpu.VMEM((E * LEAF,), jnp.float32),
            pltpu.VMEM((ROWS_PER_W,), jnp.float32),
        ] + [pltpu.SemaphoreType.DMA] * 5,
    )
    def kern(x_hbm, thr_hbm, fsel_hbm, ew_hbm, lc_hbm, out_hbm,
             x_v, thr_v, fsel_v, ew_v, lc_v, res_v,
             sem0, sem1, sem2, sem3, sem4):
        wid = lax.axis_index("s") * 2 + lax.axis_index("c")
        base = wid * ROWS_PER_W
        cps = [
            pltpu.async_copy(x_hbm.at[pl.ds(base * NVAR, ROWS_PER_W * NVAR)],
                             x_v, sem0),
            pltpu.async_copy(thr_hbm, thr_v, sem1),
            pltpu.async_copy(fsel_hbm, fsel_v, sem2),
            pltpu.async_copy(ew_hbm, ew_v, sem3),
            pltpu.async_copy(lc_hbm, lc_v, sem4),
        ]
        for cp in cps:
            cp.wait()

        def pre_round(i, _):
            sl = pl.ds(i * LANES, LANES)
            u = lax.bitcast_convert_type(x_v[sl], jnp.int32)
            u = u + 0x7FFF + (lax.shift_right_logical(u, 16) & 1)
            x_v[sl] = lax.bitcast_convert_type(u & jnp.int32(-65536),
                                               jnp.float32)
            return 0
        lax.fori_loop(0, ROWS_PER_W * NVAR // LANES, pre_round, 0)

        iota = lax.broadcasted_iota(jnp.int32, (LANES,), 0)
        zero_i = jnp.zeros((LANES,), jnp.int32)
        one_i = jnp.ones((LANES,), jnp.int32)
        rowoff = [iota * NVAR + (g * LANES * NVAR) for g in range(NGROUPS)]

        EUNROLL = 2

        def body(eh, carry):
            nxt = list(carry)
            for k in range(EUNROLL):
                e = eh * EUNROLL + k
                lbase = e * LEAF
                for g in range(NGROUPS):
                    s, a = nxt[g]
                    leaf = zero_i
                    for d in range(DEPTH):
                        idx = (e * INTERNAL + (2 ** d - 1)) + leaf
                        fi = plsc.load_gather(fsel_v, [idx])
                        tv = plsc.load_gather(thr_v, [idx])
                        xv = plsc.load_gather(x_v, [rowoff[g] + fi])
                        diff = tv - xv
                        q = diff / (1.0 + jnp.abs(diff))
                        bit = jnp.where(q > TWO_M24, zero_i, one_i)
                        leaf = leaf + leaf + bit
                    li = lbase + leaf
                    w = plsc.load_gather(ew_v, [li])
                    v = plsc.load_gather(lc_v, [li])
                    p = jnp.exp(w)
                    nxt[g] = (s + p, a + p * v)
            return tuple(nxt)

        init = tuple(
            (jnp.zeros((LANES,), jnp.float32),
             jnp.zeros((LANES,), jnp.float32))
            for _ in range(NGROUPS)
        )
        fin = lax.fori_loop(0, E // EUNROLL, body, init)
        for g in range(NGROUPS):
            s, a = fin[g]
            res_v[pl.ds(g * LANES, LANES)] = a / s
        pltpu.sync_copy(res_v, out_hbm.at[pl.ds(base, ROWS_PER_W)])

    return kern(x, thr, fsel, ew, lc)


def kernel(inputs, split_values, split_index_array, estimator_weights,
           leaf_classes_array, features_by_estimator, path_identifier_list,
           internal_node_index_list):
    sia = jax.nn.softmax(split_index_array, axis=-1)
    n_star = jnp.argmax(sia, axis=-1).astype(jnp.int32)
    thr = jnp.take_along_axis(split_values, n_star[..., None], axis=-1)[..., 0]
    fsel = jnp.take_along_axis(features_by_estimator, n_star, axis=1)

    return _forest_sc(
        inputs.reshape(-1),
        thr.reshape(-1),
        fsel.reshape(-1).astype(jnp.int32),
        estimator_weights.reshape(-1),
        leaf_classes_array.reshape(-1),
    )


# speedup vs baseline: 7.4981x; 1.2640x over previous
import functools

import jax
import jax.numpy as jnp
from jax import lax
from jax.experimental import pallas as pl
from jax.experimental.pallas import tpu as pltpu
from jax.experimental.pallas import tpu_sc as plsc

DEPTH = 5
E = 256
NVAR = 128
NSEL = 32
B = 1024
INTERNAL = 2 ** DEPTH - 1
LEAF = 2 ** DEPTH
LANES = 16
NWORKERS = 32
ROWS_PER_W = B // NWORKERS
NGROUPS = ROWS_PER_W // LANES
TWO_M24 = 5.9604644775390625e-08


def _forest_sc(x, thr, fsel, ew, lc):
    mesh = plsc.VectorSubcoreMesh(core_axis_name="c", subcore_axis_name="s")

    @functools.partial(
        pl.kernel,
        out_type=jax.ShapeDtypeStruct((B,), jnp.float32),
        mesh=mesh,
        compiler_params=pltpu.CompilerParams(needs_layout_passes=False),
        scratch_types=[
            pltpu.VMEM((ROWS_PER_W * NVAR,), jnp.float32),
            pltpu.VMEM((E * INTERNAL,), jnp.float32),
            pltpu.VMEM((E * INTERNAL,), jnp.int32),
            pltpu.VMEM((E * LEAF,), jnp.float32),
            pltpu.VMEM((E * LEAF,), jnp.float32),
            pltpu.VMEM((ROWS_PER_W,), jnp.float32),
        ] + [pltpu.SemaphoreType.DMA] * 5,
    )
    def kern(x_hbm, thr_hbm, fsel_hbm, ew_hbm, lc_hbm, out_hbm,
             x_v, thr_v, fsel_v, ew_v, lc_v, res_v,
             sem0, sem1, sem2, sem3, sem4):
        wid = lax.axis_index("s") * 2 + lax.axis_index("c")
        base = wid * ROWS_PER_W
        cps = [
            pltpu.async_copy(x_hbm.at[pl.ds(base * NVAR, ROWS_PER_W * NVAR)],
                             x_v, sem0),
            pltpu.async_copy(thr_hbm, thr_v, sem1),
            pltpu.async_copy(fsel_hbm, fsel_v, sem2),
            pltpu.async_copy(ew_hbm, ew_v, sem3),
            pltpu.async_copy(lc_hbm, lc_v, sem4),
        ]
        for cp in cps:
            cp.wait()

        def pre_round(i, _):
            sl = pl.ds(i * LANES, LANES)
            u = lax.bitcast_convert_type(x_v[sl], jnp.int32)
            u = u + 0x7FFF + (lax.shift_right_logical(u, 16) & 1)
            x_v[sl] = lax.bitcast_convert_type(u & jnp.int32(-65536),
                                               jnp.float32)
            return 0
        lax.fori_loop(0, ROWS_PER_W * NVAR // LANES, pre_round, 0)

        iota = lax.broadcasted_iota(jnp.int32, (LANES,), 0)
        zero_i = jnp.zeros((LANES,), jnp.int32)
        one_i = jnp.ones((LANES,), jnp.int32)
        rowoff = [iota * NVAR + (g * LANES * NVAR) for g in range(NGROUPS)]

        EUNROLL = 2

        def body(eh, carry):
            nxt = list(carry)
            for k in range(EUNROLL):
                e = eh * EUNROLL + k
                lbase = e * LEAF
                for g in range(NGROUPS):
                    s, a = nxt[g]
                    leaf = zero_i
                    for d in range(DEPTH):
                        idx = (e * INTERNAL + (2 ** d - 1)) + leaf
                        fi = plsc.load_gather(fsel_v, [idx])
                        tv = plsc.load_gather(thr_v, [idx])
                        xv = plsc.load_gather(x_v, [rowoff[g] + fi])
                        diff = tv - xv
                        q = diff / (1.0 + jnp.abs(diff))
                        bit = jnp.where(q > TWO_M24, zero_i, one_i)
                        leaf = leaf + leaf + bit
                    li = lbase + leaf
                    w = plsc.load_gather(ew_v, [li])
                    v = plsc.load_gather(lc_v, [li])
                    p = jnp.exp(w)
                    nxt[g] = (s + p, a + p * v)
            return tuple(nxt)

        init = tuple(
            (jnp.zeros((LANES,), jnp.float32),
             jnp.zeros((LANES,), jnp.float32))
            for _ in range(NGROUPS)
        )
        fin = lax.fori_loop(0, E // EUNROLL, body, init)
        for g in range(NGROUPS):
            s, a = fin[g]
            res_v[pl.ds(g * LANES, LANES)] = a / s
        pltpu.sync_copy(res_v, out_hbm.at[pl.ds(base, ROWS_PER_W)])

    return kern(x, thr, fsel, ew, lc)


def kernel(inputs, split_values, split_index_array, estimator_weights,
           leaf_classes_array, features_by_estimator, path_identifier_list,
           internal_node_index_list):
    sia = jax.nn.softmax(split_index_array, axis=-1)
    n_star = jnp.argmax(sia, axis=-1).astype(jnp.int32)
    oh = (lax.broadcasted_iota(jnp.int32, (1, 1, NSEL), 2)
          == n_star[..., None]).astype(jnp.float32)
    thr = jnp.sum(split_values * oh, axis=-1)
    fsel = jnp.sum(features_by_estimator[:, None, :].astype(jnp.float32) * oh,
                   axis=-1).astype(jnp.int32)

    return _forest_sc(
        inputs.reshape(-1),
        thr.reshape(-1),
        fsel.reshape(-1).astype(jnp.int32),
        estimator_weights.reshape(-1),
        leaf_classes_array.reshape(-1),
    )
